# baseline (speedup 1.0000x reference)
import jax
import jax.numpy as jnp
from jax.experimental import pallas as pl


def _node_mlp(x, W1, b1, W2, b2, interpret=False):
    N = x.shape[0]
    BN = 1000

    def body(x_ref, W1_ref, b1_ref, W2_ref, b2_ref, o_ref):
        h = jnp.maximum(
            jnp.dot(x_ref[...], W1_ref[...], preferred_element_type=jnp.float32)
            + b1_ref[...], 0.0)
        o_ref[...] = (
            jnp.dot(h, W2_ref[...], preferred_element_type=jnp.float32)
            + b2_ref[...])

    return pl.pallas_call(
        body,
        grid=(N // BN,),
        in_specs=[
            pl.BlockSpec((BN, 128), lambda i: (i, 0)),
            pl.BlockSpec((128, 128), lambda i: (0, 0)),
            pl.BlockSpec((1, 128), lambda i: (0, 0)),
            pl.BlockSpec((128, 128), lambda i: (0, 0)),
            pl.BlockSpec((1, 128), lambda i: (0, 0)),
        ],
        out_specs=pl.BlockSpec((BN, 128), lambda i: (i, 0)),
        out_shape=jax.ShapeDtypeStruct((N, 128), jnp.float32),
        interpret=interpret,
    )(x, W1, b1.reshape(1, 128), W2, b2.reshape(1, 128))


def _edge_msg(g1, ea, We1, be1, We2, be2, Wc, bc, interpret=False):
    E = g1.shape[0]
    BE = 4000

    def body(g1_ref, ea_ref, We1_ref, be1_ref, We2_ref, be2_ref, Wc_ref,
             bc_ref, o_ref):
        h = jnp.maximum(
            jnp.dot(ea_ref[...], We1_ref[...], preferred_element_type=jnp.float32)
            + be1_ref[...], 0.0)
        m2 = (jnp.dot(h, We2_ref[...], preferred_element_type=jnp.float32)
              + be2_ref[...])
        t = g1_ref[...] * m2
        o_ref[...] = jnp.tanh(
            jnp.dot(t, Wc_ref[...], preferred_element_type=jnp.float32)
            + bc_ref[...])

    return pl.pallas_call(
        body,
        grid=(E // BE,),
        in_specs=[
            pl.BlockSpec((BE, 128), lambda i: (i, 0)),
            pl.BlockSpec((BE, 16), lambda i: (i, 0)),
            pl.BlockSpec((16, 128), lambda i: (0, 0)),
            pl.BlockSpec((1, 128), lambda i: (0, 0)),
            pl.BlockSpec((128, 128), lambda i: (0, 0)),
            pl.BlockSpec((1, 128), lambda i: (0, 0)),
            pl.BlockSpec((128, 128), lambda i: (0, 0)),
            pl.BlockSpec((1, 128), lambda i: (0, 0)),
        ],
        out_specs=pl.BlockSpec((BE, 128), lambda i: (i, 0)),
        out_shape=jax.ShapeDtypeStruct((E, 128), jnp.float32),
        interpret=interpret,
    )(g1, ea, We1, be1.reshape(1, 128), We2, be2.reshape(1, 128), Wc,
      bc.reshape(1, 128))


def _edge_update(hs, hd, ea, Wu, interpret=False):
    E = hs.shape[0]
    BE = 4000

    def body(hs_ref, hd_ref, ea_ref, Wu_ref, o_ref):
        prod = hs_ref[...] * hd_ref[...]
        o_ref[...] = 0.8 * ea_ref[...] + 0.2 * jnp.dot(
            prod, Wu_ref[...], preferred_element_type=jnp.float32)

    return pl.pallas_call(
        body,
        grid=(E // BE,),
        in_specs=[
            pl.BlockSpec((BE, 128), lambda i: (i, 0)),
            pl.BlockSpec((BE, 128), lambda i: (i, 0)),
            pl.BlockSpec((BE, 16), lambda i: (i, 0)),
            pl.BlockSpec((128, 16), lambda i: (0, 0)),
        ],
        out_specs=pl.BlockSpec((BE, 16), lambda i: (i, 0)),
        out_shape=jax.ShapeDtypeStruct((E, 16), jnp.float32),
        interpret=interpret,
    )(hs, hd, ea, Wu)


def kernel(x, edge_index, edge_attr, W1, b1, W2, b2, We1, be1, We2, be2,
           Wc, bc, Wu):
    src = edge_index[0]
    dst = edge_index[1]
    node_m1 = _node_mlp(x, W1, b1, W2, b2)
    g1 = jnp.take(node_m1, src, axis=0)
    m = _edge_msg(g1, edge_attr, We1, be1, We2, be2, Wc, bc)
    agg = jnp.zeros_like(x).at[dst].add(m)
    h_new = agg + x
    hs = jnp.take(h_new, src, axis=0)
    hd = jnp.take(h_new, dst, axis=0)
    e_new = _edge_update(hs, hd, edge_attr, Wu)
    return (h_new, e_new)


# speedup vs baseline: 2.8345x; 2.8345x over previous
import functools

import jax
import jax.numpy as jnp
from jax import lax
from jax.experimental import pallas as pl
from jax.experimental.pallas import tpu as pltpu
from jax.experimental.pallas import tpu_sc as plsc

_NC = 2
_NS = 16
_NW = _NC * _NS

_C = 128
_E = 320000
_PER_W = _E // _NW
_FULL = _PER_W // _C
_REM = _PER_W - _FULL * _C
_NPAD = 10240


def _sc_gather(table, idx):
    D = table.shape[1]
    mesh = plsc.VectorSubcoreMesh(core_axis_name="c", subcore_axis_name="s")

    @functools.partial(
        pl.kernel,
        out_type=jax.ShapeDtypeStruct((_E, D), jnp.float32),
        mesh=mesh,
        scratch_types=[
            pltpu.VMEM((_C,), jnp.int32),
            pltpu.VMEM((_C, D), jnp.float32),
            pltpu.VMEM((_REM,), jnp.int32),
            pltpu.VMEM((_REM, D), jnp.float32),
            pltpu.SemaphoreType.DMA,
        ],
    )
    def k(table_hbm, idx_hbm, out_hbm, idx_v, rows_v, idx_r, rows_r, sem):
        wid = lax.axis_index("s") * _NC + lax.axis_index("c")
        base = wid * _PER_W

        def body(j, _):
            off = base + j * _C
            pltpu.sync_copy(idx_hbm.at[pl.ds(off, _C)], idx_v)
            pltpu.async_copy(table_hbm.at[idx_v], rows_v, sem).wait()
            pltpu.sync_copy(rows_v, out_hbm.at[pl.ds(off, _C)])
            return ()

        lax.fori_loop(0, _FULL, body, ())
        off = base + _FULL * _C
        pltpu.sync_copy(idx_hbm.at[pl.ds(off, _REM)], idx_r)
        pltpu.async_copy(table_hbm.at[idx_r], rows_r, sem).wait()
        pltpu.sync_copy(rows_r, out_hbm.at[pl.ds(off, _REM)])

    return k(table, idx)


def _sc_scatter_add(m, dst, zeros):
    mesh = plsc.VectorSubcoreMesh(core_axis_name="c", subcore_axis_name="s")
    rpt = _NPAD // _NS

    @functools.partial(
        pl.kernel,
        out_type=jax.ShapeDtypeStruct((_NC, _NPAD, 128), jnp.float32),
        mesh=mesh,
        scratch_types=[
            pltpu.VMEM((_C,), jnp.int32),
            pltpu.VMEM((_C, 128), jnp.float32),
            pltpu.VMEM((_REM,), jnp.int32),
            pltpu.VMEM((_REM, 128), jnp.float32),
            pltpu.VMEM_SHARED((_NPAD, 128), jnp.float32),
            pltpu.SemaphoreType.DMA,
        ],
    )
    def k(m_hbm, dst_hbm, z_hbm, out_hbm, idx_v, rows_v, idx_r, rows_r,
          agg_sh, sem):
        cid = lax.axis_index("c")
        sid = lax.axis_index("s")
        wid = sid * _NC + cid
        base = wid * _PER_W
        pltpu.sync_copy(z_hbm, agg_sh.at[pl.ds(sid * rpt, rpt)])
        plsc.subcore_barrier()

        def body(j, _):
            off = base + j * _C
            pltpu.sync_copy(dst_hbm.at[pl.ds(off, _C)], idx_v)
            pltpu.sync_copy(m_hbm.at[pl.ds(off, _C)], rows_v)
            pltpu.sync_copy(rows_v, agg_sh.at[idx_v], add=True)
            return ()

        lax.fori_loop(0, _FULL, body, ())
        off = base + _FULL * _C
        pltpu.sync_copy(dst_hbm.at[pl.ds(off, _REM)], idx_r)
        pltpu.sync_copy(m_hbm.at[pl.ds(off, _REM)], rows_r)
        pltpu.sync_copy(rows_r, agg_sh.at[idx_r], add=True)
        plsc.subcore_barrier()
        pltpu.sync_copy(agg_sh.at[pl.ds(sid * rpt, rpt)],
                        out_hbm.at[cid, pl.ds(sid * rpt, rpt)])

    return k(m, dst, zeros)


def _combine(p0, p1, x, interpret=False):
    N = x.shape[0]
    BN = 1000

    def body(p0_ref, p1_ref, x_ref, o_ref):
        o_ref[...] = p0_ref[...] + p1_ref[...] + x_ref[...]

    return pl.pallas_call(
        body,
        grid=(N // BN,),
        in_specs=[pl.BlockSpec((BN, 128), lambda i: (i, 0))] * 3,
        out_specs=pl.BlockSpec((BN, 128), lambda i: (i, 0)),
        out_shape=jax.ShapeDtypeStruct((N, 128), jnp.float32),
        interpret=interpret,
    )(p0, p1, x)


def _node_mlp(x, W1, b1, W2, b2, interpret=False):
    N = x.shape[0]
    BN = 1000

    def body(x_ref, W1_ref, b1_ref, W2_ref, b2_ref, o_ref):
        h = jnp.maximum(
            jnp.dot(x_ref[...], W1_ref[...], preferred_element_type=jnp.float32)
            + b1_ref[...], 0.0)
        o_ref[...] = (
            jnp.dot(h, W2_ref[...], preferred_element_type=jnp.float32)
            + b2_ref[...])

    return pl.pallas_call(
        body,
        grid=(N // BN,),
        in_specs=[
            pl.BlockSpec((BN, 128), lambda i: (i, 0)),
            pl.BlockSpec((128, 128), lambda i: (0, 0)),
            pl.BlockSpec((1, 128), lambda i: (0, 0)),
            pl.BlockSpec((128, 128), lambda i: (0, 0)),
            pl.BlockSpec((1, 128), lambda i: (0, 0)),
        ],
        out_specs=pl.BlockSpec((BN, 128), lambda i: (i, 0)),
        out_shape=jax.ShapeDtypeStruct((N, 128), jnp.float32),
        interpret=interpret,
    )(x, W1, b1.reshape(1, 128), W2, b2.reshape(1, 128))


def _edge_msg(g1, ea, We1, be1, We2, be2, Wc, bc, interpret=False):
    E = g1.shape[0]
    BE = 4000

    def body(g1_ref, ea_ref, We1_ref, be1_ref, We2_ref, be2_ref, Wc_ref,
             bc_ref, o_ref):
        h = jnp.maximum(
            jnp.dot(ea_ref[...], We1_ref[...], preferred_element_type=jnp.float32)
            + be1_ref[...], 0.0)
        m2 = (jnp.dot(h, We2_ref[...], preferred_element_type=jnp.float32)
              + be2_ref[...])
        t = g1_ref[...] * m2
        o_ref[...] = jnp.tanh(
            jnp.dot(t, Wc_ref[...], preferred_element_type=jnp.float32)
            + bc_ref[...])

    return pl.pallas_call(
        body,
        grid=(E // BE,),
        in_specs=[
            pl.BlockSpec((BE, 128), lambda i: (i, 0)),
            pl.BlockSpec((BE, 16), lambda i: (i, 0)),
            pl.BlockSpec((16, 128), lambda i: (0, 0)),
            pl.BlockSpec((1, 128), lambda i: (0, 0)),
            pl.BlockSpec((128, 128), lambda i: (0, 0)),
            pl.BlockSpec((1, 128), lambda i: (0, 0)),
            pl.BlockSpec((128, 128), lambda i: (0, 0)),
            pl.BlockSpec((1, 128), lambda i: (0, 0)),
        ],
        out_specs=pl.BlockSpec((BE, 128), lambda i: (i, 0)),
        out_shape=jax.ShapeDtypeStruct((E, 128), jnp.float32),
        interpret=interpret,
    )(g1, ea, We1, be1.reshape(1, 128), We2, be2.reshape(1, 128), Wc,
      bc.reshape(1, 128))


def _edge_update(hs, hd, ea, Wu, interpret=False):
    E = hs.shape[0]
    BE = 4000

    def body(hs_ref, hd_ref, ea_ref, Wu_ref, o_ref):
        prod = hs_ref[...] * hd_ref[...]
        o_ref[...] = 0.8 * ea_ref[...] + 0.2 * jnp.dot(
            prod, Wu_ref[...], preferred_element_type=jnp.float32)

    return pl.pallas_call(
        body,
        grid=(E // BE,),
        in_specs=[
            pl.BlockSpec((BE, 128), lambda i: (i, 0)),
            pl.BlockSpec((BE, 128), lambda i: (i, 0)),
            pl.BlockSpec((BE, 16), lambda i: (i, 0)),
            pl.BlockSpec((128, 16), lambda i: (0, 0)),
        ],
        out_specs=pl.BlockSpec((BE, 16), lambda i: (i, 0)),
        out_shape=jax.ShapeDtypeStruct((E, 16), jnp.float32),
        interpret=interpret,
    )(hs, hd, ea, Wu)


def kernel(x, edge_index, edge_attr, W1, b1, W2, b2, We1, be1, We2, be2,
           Wc, bc, Wu):
    src = edge_index[0]
    dst = edge_index[1]
    node_m1 = _node_mlp(x, W1, b1, W2, b2)
    g1 = _sc_gather(node_m1, src)
    m = _edge_msg(g1, edge_attr, We1, be1, We2, be2, Wc, bc)
    zeros = jnp.zeros((_NPAD // _NS, 128), jnp.float32)
    partials = _sc_scatter_add(m, dst, zeros)
    h_new = _combine(partials[0, :10000], partials[1, :10000], x)
    hs = _sc_gather(h_new, src)
    hd = _sc_gather(h_new, dst)
    e_new = _edge_update(hs, hd, edge_attr, Wu)
    return (h_new, e_new)


# speedup vs baseline: 3.7861x; 1.3357x over previous
import functools

import jax
import jax.numpy as jnp
from jax import lax
from jax.experimental import pallas as pl
from jax.experimental.pallas import tpu as pltpu
from jax.experimental.pallas import tpu_sc as plsc

_NC = 2
_NS = 16
_NW = _NC * _NS

_C = 128
_E = 320000
_PER_W = _E // _NW
_FULL = _PER_W // _C
_REM = _PER_W - _FULL * _C
_NPAD = 10240


def _sc_gather(table, idx):
    D = table.shape[1]
    NBUF = 6
    NGRP = _FULL // NBUF
    mesh = plsc.VectorSubcoreMesh(core_axis_name="c", subcore_axis_name="s")

    @functools.partial(
        pl.kernel,
        out_type=jax.ShapeDtypeStruct((_E, D), jnp.float32),
        mesh=mesh,
        scratch_types=(
            [pltpu.VMEM((_PER_W,), jnp.int32),
             pltpu.VMEM((NBUF, _C, D), jnp.float32),
             pltpu.VMEM((_REM,), jnp.int32),
             pltpu.VMEM((_REM, D), jnp.float32)]
            + [pltpu.SemaphoreType.DMA] * (2 * NBUF)
        ),
    )
    def k(table_hbm, idx_hbm, out_hbm, idx_v, rows_v, idx_r, rows_r, *sems):
        gsem = sems[:NBUF]
        wsem = sems[NBUF:]
        wid = lax.axis_index("s") * _NC + lax.axis_index("c")
        base = wid * _PER_W
        pltpu.sync_copy(idx_hbm.at[pl.ds(base, _PER_W)], idx_v)
        for b in range(NBUF):
            pltpu.async_copy(
                table_hbm.at[idx_v.at[pl.ds(b * _C, _C)]],
                rows_v.at[b], gsem[b])

        def outer(g, _):
            for b in range(NBUF):
                c = g * NBUF + b
                off = base + c * _C
                pltpu.make_async_copy(
                    table_hbm.at[pl.ds(0, _C)], rows_v.at[b],
                    gsem[b]).wait()
                pltpu.async_copy(rows_v.at[b], out_hbm.at[pl.ds(off, _C)],
                                 wsem[b])
                pltpu.make_async_copy(
                    rows_v.at[b], out_hbm.at[pl.ds(0, _C)], wsem[b]).wait()

                @pl.when(g < NGRP - 1)
                def _():
                    pltpu.async_copy(
                        table_hbm.at[idx_v.at[pl.ds((c + NBUF) * _C, _C)]],
                        rows_v.at[b], gsem[b])
            return ()

        lax.fori_loop(0, NGRP, outer, ())
        off = base + _FULL * _C
        pltpu.sync_copy(idx_hbm.at[pl.ds(off, _REM)], idx_r)
        pltpu.async_copy(table_hbm.at[idx_r], rows_r, gsem[0]).wait()
        pltpu.sync_copy(rows_r, out_hbm.at[pl.ds(off, _REM)])

    return k(table, idx)


def _sc_gather_pair(table, idx_a, idx_b):
    D = table.shape[1]
    NBUF = 3
    NGRP = _FULL // NBUF
    mesh = plsc.VectorSubcoreMesh(core_axis_name="c", subcore_axis_name="s")

    @functools.partial(
        pl.kernel,
        out_type=(jax.ShapeDtypeStruct((_E, D), jnp.float32),
                  jax.ShapeDtypeStruct((_E, D), jnp.float32)),
        mesh=mesh,
        scratch_types=(
            [pltpu.VMEM((_PER_W,), jnp.int32),
             pltpu.VMEM((_PER_W,), jnp.int32),
             pltpu.VMEM((NBUF, _C, D), jnp.float32),
             pltpu.VMEM((NBUF, _C, D), jnp.float32),
             pltpu.VMEM((_REM,), jnp.int32),
             pltpu.VMEM((_REM, D), jnp.float32)]
            + [pltpu.SemaphoreType.DMA] * (4 * NBUF)
        ),
    )
    def k(table_hbm, ia_hbm, ib_hbm, oa_hbm, ob_hbm, ia_v, ib_v, ra_v, rb_v,
          idx_r, rows_r, *sems):
        gsa = sems[:NBUF]
        gsb = sems[NBUF:2 * NBUF]
        wsa = sems[2 * NBUF:3 * NBUF]
        wsb = sems[3 * NBUF:]
        wid = lax.axis_index("s") * _NC + lax.axis_index("c")
        base = wid * _PER_W
        pltpu.sync_copy(ia_hbm.at[pl.ds(base, _PER_W)], ia_v)
        pltpu.sync_copy(ib_hbm.at[pl.ds(base, _PER_W)], ib_v)
        for b in range(NBUF):
            pltpu.async_copy(table_hbm.at[ia_v.at[pl.ds(b * _C, _C)]],
                             ra_v.at[b], gsa[b])
            pltpu.async_copy(table_hbm.at[ib_v.at[pl.ds(b * _C, _C)]],
                             rb_v.at[b], gsb[b])

        def outer(g, _):
            for b in range(NBUF):
                c = g * NBUF + b
                off = base + c * _C
                for (rv, gs, ws, oh, iv) in ((ra_v, gsa, wsa, oa_hbm, ia_v),
                                             (rb_v, gsb, wsb, ob_hbm, ib_v)):
                    pltpu.make_async_copy(
                        table_hbm.at[pl.ds(0, _C)], rv.at[b], gs[b]).wait()
                    pltpu.async_copy(rv.at[b], oh.at[pl.ds(off, _C)], ws[b])
                    pltpu.make_async_copy(
                        rv.at[b], oh.at[pl.ds(0, _C)], ws[b]).wait()

                    @pl.when(g < NGRP - 1)
                    def _():
                        pltpu.async_copy(
                            table_hbm.at[iv.at[pl.ds((c + NBUF) * _C, _C)]],
                            rv.at[b], gs[b])
            return ()

        lax.fori_loop(0, NGRP, outer, ())
        off = base + _FULL * _C
        for (ih, oh) in ((ia_hbm, oa_hbm), (ib_hbm, ob_hbm)):
            pltpu.sync_copy(ih.at[pl.ds(off, _REM)], idx_r)
            pltpu.async_copy(table_hbm.at[idx_r], rows_r, gsa[0]).wait()
            pltpu.sync_copy(rows_r, oh.at[pl.ds(off, _REM)])

    return k(table, idx_a, idx_b)


def _sc_scatter_add(m, dst, zeros):
    mesh = plsc.VectorSubcoreMesh(core_axis_name="c", subcore_axis_name="s")
    rpt = _NPAD // _NS

    NBUF = 2
    NGRP = _FULL // NBUF

    @functools.partial(
        pl.kernel,
        out_type=jax.ShapeDtypeStruct((_NC, _NPAD, 128), jnp.float32),
        mesh=mesh,
        scratch_types=(
            [pltpu.VMEM((NBUF, _C), jnp.int32),
             pltpu.VMEM((NBUF, _C, 128), jnp.float32),
             pltpu.VMEM((_REM,), jnp.int32),
             pltpu.VMEM((_REM, 128), jnp.float32),
             pltpu.VMEM_SHARED((_NPAD, 128), jnp.float32)]
            + [pltpu.SemaphoreType.DMA] * (3 * NBUF)
        ),
    )
    def k(m_hbm, dst_hbm, z_hbm, out_hbm, idx_v, rows_v, idx_r, rows_r,
          agg_sh, *sems):
        isem = sems[:NBUF]
        lsem = sems[NBUF:2 * NBUF]
        ssem = sems[2 * NBUF:]
        cid = lax.axis_index("c")
        sid = lax.axis_index("s")
        wid = sid * _NC + cid
        base = wid * _PER_W
        pltpu.sync_copy(z_hbm, agg_sh.at[pl.ds(sid * rpt, rpt)])
        plsc.subcore_barrier()
        for b in range(NBUF):
            off = base + b * _C
            pltpu.async_copy(dst_hbm.at[pl.ds(off, _C)], idx_v.at[b],
                             isem[b])
            pltpu.async_copy(m_hbm.at[pl.ds(off, _C)], rows_v.at[b],
                             lsem[b])

        def outer(g, _):
            for b in range(NBUF):
                c = g * NBUF + b
                pltpu.make_async_copy(dst_hbm.at[pl.ds(0, _C)],
                                      idx_v.at[b], isem[b]).wait()
                pltpu.make_async_copy(m_hbm.at[pl.ds(0, _C)],
                                      rows_v.at[b], lsem[b]).wait()
                pltpu.async_copy(rows_v.at[b], agg_sh.at[idx_v.at[b]],
                                 ssem[b], add=True)
                pltpu.make_async_copy(rows_v.at[b], agg_sh.at[idx_v.at[b]],
                                      ssem[b]).wait()

                @pl.when(g < NGRP - 1)
                def _():
                    off = base + (c + NBUF) * _C
                    pltpu.async_copy(dst_hbm.at[pl.ds(off, _C)],
                                     idx_v.at[b], isem[b])
                    pltpu.async_copy(m_hbm.at[pl.ds(off, _C)],
                                     rows_v.at[b], lsem[b])
            return ()

        lax.fori_loop(0, NGRP, outer, ())
        off = base + _FULL * _C
        pltpu.sync_copy(dst_hbm.at[pl.ds(off, _REM)], idx_r)
        pltpu.sync_copy(m_hbm.at[pl.ds(off, _REM)], rows_r)
        pltpu.sync_copy(rows_r, agg_sh.at[idx_r], add=True)
        plsc.subcore_barrier()
        pltpu.sync_copy(agg_sh.at[pl.ds(sid * rpt, rpt)],
                        out_hbm.at[cid, pl.ds(sid * rpt, rpt)])

    return k(m, dst, zeros)


def _combine(p0, p1, x, interpret=False):
    N = x.shape[0]
    BN = 1000

    def body(p0_ref, p1_ref, x_ref, o_ref):
        o_ref[...] = p0_ref[...] + p1_ref[...] + x_ref[...]

    return pl.pallas_call(
        body,
        grid=(N // BN,),
        in_specs=[pl.BlockSpec((BN, 128), lambda i: (i, 0))] * 3,
        out_specs=pl.BlockSpec((BN, 128), lambda i: (i, 0)),
        out_shape=jax.ShapeDtypeStruct((N, 128), jnp.float32),
        interpret=interpret,
    )(p0, p1, x)


def _node_mlp(x, W1, b1, W2, b2, interpret=False):
    N = x.shape[0]
    BN = 1000

    def body(x_ref, W1_ref, b1_ref, W2_ref, b2_ref, o_ref):
        h = jnp.maximum(
            jnp.dot(x_ref[...], W1_ref[...], preferred_element_type=jnp.float32)
            + b1_ref[...], 0.0)
        o_ref[...] = (
            jnp.dot(h, W2_ref[...], preferred_element_type=jnp.float32)
            + b2_ref[...])

    return pl.pallas_call(
        body,
        grid=(N // BN,),
        in_specs=[
            pl.BlockSpec((BN, 128), lambda i: (i, 0)),
            pl.BlockSpec((128, 128), lambda i: (0, 0)),
            pl.BlockSpec((1, 128), lambda i: (0, 0)),
            pl.BlockSpec((128, 128), lambda i: (0, 0)),
            pl.BlockSpec((1, 128), lambda i: (0, 0)),
        ],
        out_specs=pl.BlockSpec((BN, 128), lambda i: (i, 0)),
        out_shape=jax.ShapeDtypeStruct((N, 128), jnp.float32),
        interpret=interpret,
    )(x, W1, b1.reshape(1, 128), W2, b2.reshape(1, 128))


def _edge_msg(g1, ea, We1, be1, We2, be2, Wc, bc, interpret=False):
    E = g1.shape[0]
    BE = 4000

    def body(g1_ref, ea_ref, We1_ref, be1_ref, We2_ref, be2_ref, Wc_ref,
             bc_ref, o_ref):
        h = jnp.maximum(
            jnp.dot(ea_ref[...], We1_ref[...], preferred_element_type=jnp.float32)
            + be1_ref[...], 0.0)
        m2 = (jnp.dot(h, We2_ref[...], preferred_element_type=jnp.float32)
              + be2_ref[...])
        t = g1_ref[...] * m2
        o_ref[...] = jnp.tanh(
            jnp.dot(t, Wc_ref[...], preferred_element_type=jnp.float32)
            + bc_ref[...])

    return pl.pallas_call(
        body,
        grid=(E // BE,),
        in_specs=[
            pl.BlockSpec((BE, 128), lambda i: (i, 0)),
            pl.BlockSpec((BE, 16), lambda i: (i, 0)),
            pl.BlockSpec((16, 128), lambda i: (0, 0)),
            pl.BlockSpec((1, 128), lambda i: (0, 0)),
            pl.BlockSpec((128, 128), lambda i: (0, 0)),
            pl.BlockSpec((1, 128), lambda i: (0, 0)),
            pl.BlockSpec((128, 128), lambda i: (0, 0)),
            pl.BlockSpec((1, 128), lambda i: (0, 0)),
        ],
        out_specs=pl.BlockSpec((BE, 128), lambda i: (i, 0)),
        out_shape=jax.ShapeDtypeStruct((E, 128), jnp.float32),
        interpret=interpret,
    )(g1, ea, We1, be1.reshape(1, 128), We2, be2.reshape(1, 128), Wc,
      bc.reshape(1, 128))


def _edge_update(hs, hd, ea, Wu, interpret=False):
    E = hs.shape[0]
    BE = 4000

    def body(hs_ref, hd_ref, ea_ref, Wu_ref, o_ref):
        prod = hs_ref[...] * hd_ref[...]
        o_ref[...] = 0.8 * ea_ref[...] + 0.2 * jnp.dot(
            prod, Wu_ref[...], preferred_element_type=jnp.float32)

    return pl.pallas_call(
        body,
        grid=(E // BE,),
        in_specs=[
            pl.BlockSpec((BE, 128), lambda i: (i, 0)),
            pl.BlockSpec((BE, 128), lambda i: (i, 0)),
            pl.BlockSpec((BE, 16), lambda i: (i, 0)),
            pl.BlockSpec((128, 16), lambda i: (0, 0)),
        ],
        out_specs=pl.BlockSpec((BE, 16), lambda i: (i, 0)),
        out_shape=jax.ShapeDtypeStruct((E, 16), jnp.float32),
        interpret=interpret,
    )(hs, hd, ea, Wu)


def kernel(x, edge_index, edge_attr, W1, b1, W2, b2, We1, be1, We2, be2,
           Wc, bc, Wu):
    src = edge_index[0]
    dst = edge_index[1]
    node_m1 = _node_mlp(x, W1, b1, W2, b2)
    g1 = _sc_gather(node_m1, src)
    m = _edge_msg(g1, edge_attr, We1, be1, We2, be2, Wc, bc)
    zeros = jnp.zeros((_NPAD // _NS, 128), jnp.float32)
    partials = _sc_scatter_add(m, dst, zeros)
    h_new = _combine(partials[0, :10000], partials[1, :10000], x)
    hs, hd = _sc_gather_pair(h_new, src, dst)
    e_new = _edge_update(hs, hd, edge_attr, Wu)
    return (h_new, e_new)
